# Initial kernel scaffold; baseline (speedup 1.0000x reference)
import jax
import jax.numpy as jnp
from jax.experimental import pallas as pl


def kernel(x, cb_topic_0, cb_topic_1, cb_style_2, cb_style_3, temperature):
    raise NotImplementedError("write your pallas kernel here")


# speedup vs baseline: 2.8802x; 2.8802x over previous
import functools

import jax
import jax.numpy as jnp
from jax.experimental import pallas as pl
from jax.experimental.pallas import tpu as pltpu

B, T, HIDDEN = 8, 576, 1024
NUM_LAYERS = 4
LAYER_DIM = HIDDEN // NUM_LAYERS
CB_SIZE = 1024
N = B * T
TILE = 256


def _body(inv_t_ref, x_ref, cbt_ref, cb_ref,
          quant_ref, qb0_ref, qb1_ref, qb2_ref, qb3_ref, idx_ref, probs_ref):
    inv_t = inv_t_ref[0, 0]
    qb_refs = (qb0_ref, qb1_ref, qb2_ref, qb3_ref)
    probs_acc = jnp.zeros((TILE, CB_SIZE), jnp.float32)
    for l in range(NUM_LAYERS):
        xb = x_ref[:, l * LAYER_DIM:(l + 1) * LAYER_DIM]
        cbt = cbt_ref[l]
        cb = cb_ref[l]
        sim = jnp.dot(xb, cbt, preferred_element_type=jnp.float32) * inv_t
        m = jnp.max(sim, axis=1, keepdims=True)
        e = jnp.exp(sim - m)
        p = e / jnp.sum(e, axis=1, keepdims=True)
        probs_acc = probs_acc + p
        iota = jax.lax.broadcasted_iota(jnp.int32, sim.shape, 1)
        idx = jnp.min(jnp.where(sim == m, iota, CB_SIZE), axis=1)
        idx_ref[l, :] = idx
        onehot = (iota == idx[:, None]).astype(jnp.float32)
        hard = jnp.dot(onehot, cb, preferred_element_type=jnp.float32)
        qb_refs[l][...] = hard
        quant_ref[:, l * LAYER_DIM:(l + 1) * LAYER_DIM] = hard
    probs_ref[...] = probs_acc * 0.25


@jax.jit
def _run(x2d, cbt, cb, inv_t):
    out_shapes = (
        jax.ShapeDtypeStruct((N, HIDDEN), jnp.float32),
        jax.ShapeDtypeStruct((N, LAYER_DIM), jnp.float32),
        jax.ShapeDtypeStruct((N, LAYER_DIM), jnp.float32),
        jax.ShapeDtypeStruct((N, LAYER_DIM), jnp.float32),
        jax.ShapeDtypeStruct((N, LAYER_DIM), jnp.float32),
        jax.ShapeDtypeStruct((NUM_LAYERS, N), jnp.int32),
        jax.ShapeDtypeStruct((N, CB_SIZE), jnp.float32),
    )
    grid = (N // TILE,)
    in_specs = [
        pl.BlockSpec((1, 1), lambda i: (0, 0), memory_space=pltpu.SMEM),
        pl.BlockSpec((NUM_LAYERS, LAYER_DIM, CB_SIZE), lambda i: (0, 0, 0)),
        pl.BlockSpec((NUM_LAYERS, CB_SIZE, LAYER_DIM), lambda i: (0, 0, 0)),
    ]
    out_specs = (
        pl.BlockSpec((TILE, HIDDEN), lambda i: (i, 0)),
        pl.BlockSpec((TILE, LAYER_DIM), lambda i: (i, 0)),
        pl.BlockSpec((TILE, LAYER_DIM), lambda i: (i, 0)),
        pl.BlockSpec((TILE, LAYER_DIM), lambda i: (i, 0)),
        pl.BlockSpec((TILE, LAYER_DIM), lambda i: (i, 0)),
        pl.BlockSpec((NUM_LAYERS, TILE), lambda i: (0, i)),
        pl.BlockSpec((TILE, CB_SIZE), lambda i: (i, 0)),
    )
    return pl.pallas_call(
        _body,
        grid=grid,
        in_specs=[in_specs[0],
                  pl.BlockSpec((TILE, HIDDEN), lambda i: (i, 0)),
                  in_specs[1], in_specs[2]],
        out_specs=out_specs,
        out_shape=out_shapes,
    )(inv_t, x2d, cbt, cb)


def kernel(x, cb_topic_0, cb_topic_1, cb_style_2, cb_style_3, temperature):
    codebooks = (cb_topic_0, cb_topic_1, cb_style_2, cb_style_3)
    temp = jnp.maximum(temperature, 0.04)
    inv_t = (1.0 / temp).reshape(1, 1).astype(jnp.float32)
    x2d = x.reshape(N, HIDDEN)
    cb = jnp.stack(codebooks)
    cbt = jnp.stack([c.T for c in codebooks])
    quant, qb0, qb1, qb2, qb3, idx_all, probs = _run(x2d, cbt, cb, inv_t)
    quantized = quant.reshape(B, T, HIDDEN)
    indices = tuple(idx_all[l].reshape(B, T) for l in range(NUM_LAYERS))
    qblocks = tuple(q.reshape(B, T, LAYER_DIM) for q in (qb0, qb1, qb2, qb3))
    avg_code_probs = probs.reshape(B, T, CB_SIZE)
    return (quantized, indices, qblocks, avg_code_probs, x)


# speedup vs baseline: 2.9749x; 1.0329x over previous
import jax
import jax.numpy as jnp
from jax.experimental import pallas as pl
from jax.experimental.pallas import tpu as pltpu

B, T, HIDDEN = 8, 576, 1024
NUM_LAYERS = 4
LAYER_DIM = HIDDEN // NUM_LAYERS
CB_SIZE = 1024
N = B * T
TILE = 256


def _body(x_ref, cbt_ref, cb_ref,
          quant_ref, qb0_ref, qb1_ref, qb2_ref, qb3_ref, idx_ref, probs_ref):
    qb_refs = (qb0_ref, qb1_ref, qb2_ref, qb3_ref)
    probs_acc = jnp.zeros((TILE, CB_SIZE), jnp.float32)
    idx_cols = []
    for l in range(NUM_LAYERS):
        xb = x_ref[:, l * LAYER_DIM:(l + 1) * LAYER_DIM]
        sim = jnp.dot(xb, cbt_ref[l], preferred_element_type=jnp.float32)
        m = jnp.max(sim, axis=1, keepdims=True)
        e = jnp.exp(sim - m)
        s = jnp.sum(e, axis=1, keepdims=True)
        probs_acc = probs_acc + e * (0.25 / s)
        iota = jax.lax.broadcasted_iota(jnp.int32, sim.shape, 1)
        idxk = jnp.min(jnp.where(sim == m, iota, CB_SIZE), axis=1, keepdims=True)
        idx_cols.append(idxk)
        onehot = (iota == idxk).astype(jnp.float32)
        hard = jnp.dot(onehot, cb_ref[l], preferred_element_type=jnp.float32)
        qb_refs[l][...] = hard
        quant_ref[:, l * LAYER_DIM:(l + 1) * LAYER_DIM] = hard
    idx_ref[...] = jnp.concatenate(idx_cols, axis=1)
    probs_ref[...] = probs_acc


@jax.jit
def _run(x2d, cbt, cb):
    out_shapes = (
        jax.ShapeDtypeStruct((N, HIDDEN), jnp.float32),
        jax.ShapeDtypeStruct((N, LAYER_DIM), jnp.float32),
        jax.ShapeDtypeStruct((N, LAYER_DIM), jnp.float32),
        jax.ShapeDtypeStruct((N, LAYER_DIM), jnp.float32),
        jax.ShapeDtypeStruct((N, LAYER_DIM), jnp.float32),
        jax.ShapeDtypeStruct((N, NUM_LAYERS), jnp.int32),
        jax.ShapeDtypeStruct((N, CB_SIZE), jnp.float32),
    )
    grid = (N // TILE,)
    out_specs = (
        pl.BlockSpec((TILE, HIDDEN), lambda i: (i, 0)),
        pl.BlockSpec((TILE, LAYER_DIM), lambda i: (i, 0)),
        pl.BlockSpec((TILE, LAYER_DIM), lambda i: (i, 0)),
        pl.BlockSpec((TILE, LAYER_DIM), lambda i: (i, 0)),
        pl.BlockSpec((TILE, LAYER_DIM), lambda i: (i, 0)),
        pl.BlockSpec((TILE, NUM_LAYERS), lambda i: (i, 0)),
        pl.BlockSpec((TILE, CB_SIZE), lambda i: (i, 0)),
    )
    return pl.pallas_call(
        _body,
        grid=grid,
        in_specs=[pl.BlockSpec((TILE, HIDDEN), lambda i: (i, 0)),
                  pl.BlockSpec((NUM_LAYERS, LAYER_DIM, CB_SIZE), lambda i: (0, 0, 0)),
                  pl.BlockSpec((NUM_LAYERS, CB_SIZE, LAYER_DIM), lambda i: (0, 0, 0))],
        out_specs=out_specs,
        out_shape=out_shapes,
    )(x2d, cbt, cb)


def kernel(x, cb_topic_0, cb_topic_1, cb_style_2, cb_style_3, temperature):
    codebooks = (cb_topic_0, cb_topic_1, cb_style_2, cb_style_3)
    temp = jnp.maximum(temperature, 0.04)
    inv_t = (1.0 / temp).astype(jnp.float32)
    x2d = x.reshape(N, HIDDEN)
    cb = jnp.stack(codebooks)
    cbt = jnp.stack([c.T for c in codebooks]) * inv_t
    quant, qb0, qb1, qb2, qb3, idx_all, probs = _run(x2d, cbt, cb)
    quantized = quant.reshape(B, T, HIDDEN)
    indices = tuple(idx_all[:, l].reshape(B, T) for l in range(NUM_LAYERS))
    qblocks = tuple(q.reshape(B, T, LAYER_DIM) for q in (qb0, qb1, qb2, qb3))
    avg_code_probs = probs.reshape(B, T, CB_SIZE)
    return (quantized, indices, qblocks, avg_code_probs, x)
